# baseline (speedup 1.0000x reference)
import jax
import jax.numpy as jnp
from jax import lax
from jax.experimental import pallas as pl
from jax.experimental.pallas import tpu as pltpu
from jax.experimental.pallas import tpu_sc as plsc

U = 100000
V = 1000000
D = 32
B = 16384
NNEG = 50

NC = 2
NS = 16
NW = NC * NS
CH = 16
PER_W = B // NW
NCHUNK = PER_W // CH
NROWS = CH * NNEG
NGRP = (NNEG + 15) // 16
SCHUNK = 128


def _tec_body(users_hbm, items_i_hbm, items_j_hbm, negs_hbm, h_hbm, p_hbm,
              q_hbm, pos_hbm, neg_hbm, idx_u_w, idx_i_w, idx_j_w,
              idx_n0, idx_n1, hu0, hu1, pi0, pi1, qj0, qj1,
              negp0, negp1, negq0, negq1, pos_out, neg_out, sem0, sem1):
    wid = lax.axis_index("s") * NC + lax.axis_index("c")
    wbase = wid * PER_W
    iota = lax.iota(jnp.int32, 16)

    idx_n = (idx_n0, idx_n1)
    hu = (hu0, hu1)
    pi = (pi0, pi1)
    qj = (qj0, qj1)
    negp = (negp0, negp1)
    negq = (negq0, negq1)
    sems = (sem0, sem1)

    pltpu.sync_copy(users_hbm.at[pl.ds(wbase, PER_W)], idx_u_w)
    pltpu.sync_copy(items_i_hbm.at[pl.ds(wbase, PER_W)], idx_i_w)
    pltpu.sync_copy(items_j_hbm.at[pl.ds(wbase, PER_W)], idx_j_w)

    def fire(t, s):
        nb = (wbase + t * CH) * NNEG
        pltpu.sync_copy(negs_hbm.at[pl.ds(nb, NROWS)], idx_n[s])
        pltpu.async_copy(h_hbm.at[idx_u_w.at[pl.ds(t * CH, CH)]], hu[s], sems[s])
        pltpu.async_copy(p_hbm.at[idx_i_w.at[pl.ds(t * CH, CH)]], pi[s], sems[s])
        pltpu.async_copy(q_hbm.at[idx_j_w.at[pl.ds(t * CH, CH)]], qj[s], sems[s])
        for off in range(0, NROWS, SCHUNK):
            w = min(SCHUNK, NROWS - off)
            pltpu.async_copy(p_hbm.at[idx_n[s].at[pl.ds(off, w)]],
                             negp[s].at[pl.ds(off, w)], sems[s])
            pltpu.async_copy(q_hbm.at[idx_n[s].at[pl.ds(off, w)]],
                             negq[s].at[pl.ds(off, w)], sems[s])

    def drain(s):
        pltpu.make_async_copy(h_hbm.at[idx_u_w.at[pl.ds(0, CH)]], hu[s], sems[s]).wait()
        pltpu.make_async_copy(p_hbm.at[idx_i_w.at[pl.ds(0, CH)]], pi[s], sems[s]).wait()
        pltpu.make_async_copy(q_hbm.at[idx_j_w.at[pl.ds(0, CH)]], qj[s], sems[s]).wait()
        for off in range(0, NROWS, SCHUNK):
            w = min(SCHUNK, NROWS - off)
            pltpu.make_async_copy(p_hbm.at[idx_n[s].at[pl.ds(off, w)]],
                                  negp[s].at[pl.ds(off, w)], sems[s]).wait()
            pltpu.make_async_copy(q_hbm.at[idx_n[s].at[pl.ds(off, w)]],
                                  negq[s].at[pl.ds(off, w)], sems[s]).wait()

    def compute(t, s):
        base = wbase + t * CH
        acc = jnp.zeros((16,), jnp.float32)
        for d in range(D):
            colv = (iota + d) & (D - 1)
            hv = plsc.load_gather(hu[s], [iota, colv])
            pv = plsc.load_gather(pi[s], [iota, colv])
            qv = plsc.load_gather(qj[s], [iota, colv])
            acc = acc + hv * (pv + qv)
        pos_out[...] = acc
        pltpu.sync_copy(pos_out, pos_hbm.at[pl.ds(base, CH)])

        def elem_body(b, ecarry):
            rows = [jnp.minimum(b * NNEG + g * 16 + iota, NROWS - 1)
                    for g in range(NGRP)]
            bv = jnp.full((16,), 0, jnp.int32) + b
            accs = [jnp.zeros((16,), jnp.float32) for _ in range(NGRP)]
            for d in range(D):
                colv = (iota + d) & (D - 1)
                hv = plsc.load_gather(hu[s], [bv, colv])
                for g in range(NGRP):
                    pvv = plsc.load_gather(negp[s], [rows[g], colv])
                    qvv = plsc.load_gather(negq[s], [rows[g], colv])
                    accs[g] = accs[g] + hv * (pvv + qvv)
            for g in range(NGRP):
                lane_n = g * 16 + iota
                plsc.store_scatter(neg_out, [b * NNEG + lane_n], accs[g],
                                   mask=lane_n < NNEG)
            return ecarry

        lax.fori_loop(0, CH, elem_body, 0)
        pltpu.sync_copy(neg_out, neg_hbm.at[pl.ds(base * NNEG, NROWS)])

    fire(0, 0)
    fire(1, 1)

    def body2(i, carry):
        tt = i * 2
        for s in (0, 1):
            t = tt + s
            drain(s)
            compute(t, s)

            @pl.when(t + 2 < NCHUNK)
            def _():
                fire(t + 2, s)
        return carry

    lax.fori_loop(0, NCHUNK // 2, body2, 0)


@jax.jit
def _run(users, items_i, items_j, negs_flat, h, p, q):
    mesh = plsc.VectorSubcoreMesh(core_axis_name="c", subcore_axis_name="s")
    f = pl.kernel(
        _tec_body,
        out_type=(
            jax.ShapeDtypeStruct((B,), jnp.float32),
            jax.ShapeDtypeStruct((B * NNEG,), jnp.float32),
        ),
        mesh=mesh,
        compiler_params=pltpu.CompilerParams(needs_layout_passes=False,
                                             use_tc_tiling_on_sc=False),
        scratch_types=(
            pltpu.VMEM((PER_W,), jnp.int32),
            pltpu.VMEM((PER_W,), jnp.int32),
            pltpu.VMEM((PER_W,), jnp.int32),
            pltpu.VMEM((NROWS,), jnp.int32),
            pltpu.VMEM((NROWS,), jnp.int32),
            pltpu.VMEM((CH, D), jnp.float32),
            pltpu.VMEM((CH, D), jnp.float32),
            pltpu.VMEM((CH, D), jnp.float32),
            pltpu.VMEM((CH, D), jnp.float32),
            pltpu.VMEM((CH, D), jnp.float32),
            pltpu.VMEM((CH, D), jnp.float32),
            pltpu.VMEM((NROWS, D), jnp.float32),
            pltpu.VMEM((NROWS, D), jnp.float32),
            pltpu.VMEM((NROWS, D), jnp.float32),
            pltpu.VMEM((NROWS, D), jnp.float32),
            pltpu.VMEM((CH,), jnp.float32),
            pltpu.VMEM((NROWS,), jnp.float32),
            pltpu.SemaphoreType.DMA,
            pltpu.SemaphoreType.DMA,
        ),
    )
    return f(users, items_i, items_j, negs_flat, h, p, q)


def kernel(users, items_i, items_j, negs, h, p, q):
    pos, neg_flat = _run(users.astype(jnp.int32), items_i, items_j,
                         negs.reshape(B * NNEG), h, p, q)
    return pos, neg_flat.reshape(B, NNEG)


# speedup vs baseline: 1.3127x; 1.3127x over previous
import jax
import jax.numpy as jnp
from jax import lax
from jax.experimental import pallas as pl
from jax.experimental.pallas import tpu as pltpu
from jax.experimental.pallas import tpu_sc as plsc

U = 100000
V = 1000000
D = 32
B = 16384
NNEG = 50

NC = 2
NS = 16
NW = NC * NS
CH = 16
PER_W = B // NW
NCHUNK = PER_W // CH
NROWS = CH * NNEG
NGRP = (NNEG + 15) // 16
SCHUNK = 128


def _tec_body(negs_hbm, s_hbm, hu_hbm, pi_hbm, qj_hbm, pos_hbm, neg_hbm,
              idx_n0, idx_n1, sr0, sr1, hu_v, pi_v, qj_v,
              pos_out, neg_out, sem0, sem1):
    wid = lax.axis_index("s") * NC + lax.axis_index("c")
    wbase = wid * PER_W
    iota = lax.iota(jnp.int32, 16)

    idx_n = (idx_n0, idx_n1)
    sr = (sr0, sr1)
    sems = (sem0, sem1)

    def fire(t, s):
        nb = (wbase + t * CH) * NNEG
        pltpu.sync_copy(negs_hbm.at[pl.ds(nb, NROWS)], idx_n[s])
        for off in range(0, NROWS, SCHUNK):
            w = min(SCHUNK, NROWS - off)
            pltpu.async_copy(s_hbm.at[idx_n[s].at[pl.ds(off, w)]],
                             sr[s].at[pl.ds(off, w)], sems[s])

    def drain(s):
        for off in range(0, NROWS, SCHUNK):
            w = min(SCHUNK, NROWS - off)
            pltpu.make_async_copy(s_hbm.at[idx_n[s].at[pl.ds(off, w)]],
                                  sr[s].at[pl.ds(off, w)], sems[s]).wait()

    def compute(t, s):
        base = wbase + t * CH
        pltpu.sync_copy(hu_hbm.at[pl.ds(base, CH)], hu_v)
        pltpu.sync_copy(pi_hbm.at[pl.ds(base, CH)], pi_v)
        pltpu.sync_copy(qj_hbm.at[pl.ds(base, CH)], qj_v)

        acc = jnp.zeros((16,), jnp.float32)
        for d in range(D):
            colv = (iota + d) & (D - 1)
            hv = plsc.load_gather(hu_v, [iota, colv])
            pv = plsc.load_gather(pi_v, [iota, colv])
            qv = plsc.load_gather(qj_v, [iota, colv])
            acc = acc + hv * (pv + qv)
        pos_out[...] = acc
        pltpu.sync_copy(pos_out, pos_hbm.at[pl.ds(base, CH)])

        def elem_body(b, ecarry):
            rows = [jnp.minimum(b * NNEG + g * 16 + iota, NROWS - 1)
                    for g in range(NGRP)]
            bv = jnp.full((16,), 0, jnp.int32) + b
            accs = [jnp.zeros((16,), jnp.float32) for _ in range(NGRP)]
            for d in range(D):
                colv = (iota + d) & (D - 1)
                hv = plsc.load_gather(hu_v, [bv, colv])
                for g in range(NGRP):
                    svv = plsc.load_gather(sr[s], [rows[g], colv])
                    accs[g] = accs[g] + hv * svv
            for g in range(NGRP):
                lane_n = g * 16 + iota
                plsc.store_scatter(neg_out, [b * NNEG + lane_n], accs[g],
                                   mask=lane_n < NNEG)
            return ecarry

        lax.fori_loop(0, CH, elem_body, 0)
        pltpu.sync_copy(neg_out, neg_hbm.at[pl.ds(base * NNEG, NROWS)])

    fire(0, 0)
    fire(1, 1)

    def body2(i, carry):
        tt = i * 2
        for s in (0, 1):
            t = tt + s
            drain(s)
            compute(t, s)

            @pl.when(t + 2 < NCHUNK)
            def _():
                fire(t + 2, s)
        return carry

    lax.fori_loop(0, NCHUNK // 2, body2, 0)


@jax.jit
def _run(negs_flat, s_tab, hu_g, pi_g, qj_g):
    mesh = plsc.VectorSubcoreMesh(core_axis_name="c", subcore_axis_name="s")
    f = pl.kernel(
        _tec_body,
        out_type=(
            jax.ShapeDtypeStruct((B,), jnp.float32),
            jax.ShapeDtypeStruct((B * NNEG,), jnp.float32),
        ),
        mesh=mesh,
        compiler_params=pltpu.CompilerParams(needs_layout_passes=False,
                                             use_tc_tiling_on_sc=False),
        scratch_types=(
            pltpu.VMEM((NROWS,), jnp.int32),
            pltpu.VMEM((NROWS,), jnp.int32),
            pltpu.VMEM((NROWS, D), jnp.float32),
            pltpu.VMEM((NROWS, D), jnp.float32),
            pltpu.VMEM((CH, D), jnp.float32),
            pltpu.VMEM((CH, D), jnp.float32),
            pltpu.VMEM((CH, D), jnp.float32),
            pltpu.VMEM((CH,), jnp.float32),
            pltpu.VMEM((NROWS,), jnp.float32),
            pltpu.SemaphoreType.DMA,
            pltpu.SemaphoreType.DMA,
        ),
    )
    return f(negs_flat, s_tab, hu_g, pi_g, qj_g)


def kernel(users, items_i, items_j, negs, h, p, q):
    s_tab = p + q
    hu_g = jnp.take(h, users.astype(jnp.int32), axis=0)
    pi_g = jnp.take(p, items_i, axis=0)
    qj_g = jnp.take(q, items_j, axis=0)
    pos, neg_flat = _run(negs.reshape(B * NNEG), s_tab, hu_g, pi_g, qj_g)
    return pos, neg_flat.reshape(B, NNEG)
